# baseline (speedup 1.0000x reference)
import functools

import jax
import jax.numpy as jnp
from jax import lax
from jax.experimental import pallas as pl
from jax.experimental.pallas import tpu as pltpu
from jax.experimental.pallas import tpu_sc as plsc

_L = 16


def _merge16(a, b):
    rb = lax.rev(b, (0,))
    lo = jnp.minimum(a, rb)
    hi = jnp.maximum(a, rb)
    return lax.sort(lo), lax.sort(hi)


def _merge32(a0, a1, b0, b1):
    rb0 = lax.rev(b1, (0,))
    rb1 = lax.rev(b0, (0,))
    lo0 = jnp.minimum(a0, rb0)
    lo1 = jnp.minimum(a1, rb1)
    hi0 = jnp.maximum(a0, rb0)
    hi1 = jnp.maximum(a1, rb1)
    p0 = jnp.minimum(lo0, lo1)
    p1 = jnp.maximum(lo0, lo1)
    q0 = jnp.minimum(hi0, hi1)
    q1 = jnp.maximum(hi0, hi1)
    return lax.sort(p0), lax.sort(p1), lax.sort(q0), lax.sort(q1)


@functools.partial(
    pl.kernel,
    mesh=plsc.VectorSubcoreMesh(core_axis_name="c", subcore_axis_name="s"),
    out_type=jax.ShapeDtypeStruct((4 * _L,), jnp.float32),
    scratch_types=[
        pltpu.VMEM((4 * _L,), jnp.float32),
        pltpu.VMEM((4 * _L,), jnp.float32),
    ],
    compiler_params=pltpu.CompilerParams(needs_layout_passes=False),
)
def _sc_sigmoid_sort(raw_hbm, out_hbm, x_v, o_v):
    is_w0 = jnp.logical_and(lax.axis_index("c") == 0, lax.axis_index("s") == 0)

    @pl.when(is_w0)
    def _():
        pltpu.sync_copy(raw_hbm, x_v)
        v = [x_v[pl.ds(i * _L, _L)] for i in range(4)]
        v = [1.0 / (1.0 + jnp.exp(-u)) for u in v]
        lane = lax.iota(jnp.int32, _L)
        v[3] = jnp.where(lane == _L - 1, jnp.float32(jnp.inf), v[3])
        s = [lax.sort(u) for u in v]
        a0, a1 = _merge16(s[0], s[1])
        b0, b1 = _merge16(s[2], s[3])
        f = _merge32(a0, a1, b0, b1)
        for i in range(4):
            o_v[pl.ds(i * _L, _L)] = f[i]
        pltpu.sync_copy(o_v, out_hbm)


@jax.jit
def kernel(raw):
    padded = jnp.pad(raw, (0, 1))
    out = _sc_sigmoid_sort(padded)
    return out[:63]
